# Initial kernel scaffold; baseline (speedup 1.0000x reference)
import jax
import jax.numpy as jnp
from jax.experimental import pallas as pl


def kernel(x, edge_index, W1, b1, W2, b2):
    raise NotImplementedError("write your pallas kernel here")


# speedup vs baseline: 31.2743x; 31.2743x over previous
import functools

import jax
import jax.numpy as jnp
from jax import lax
from jax.experimental import pallas as pl
from jax.experimental.pallas import tpu as pltpu
from jax.experimental.pallas import tpu_sc as plsc

N = 10000
D = 128
H = 16
C = 2
NC = 2
NS = 16
NW = NC * NS
CB = 128
NROW = N + 112
RPT = NROW // NS


def _sc_mesh():
    return plsc.VectorSubcoreMesh(core_axis_name="c", subcore_axis_name="s")


def _make_deg_kernel(nch):
    def body(dst_hbm, zero_hbm, out_hbm, dstb, ones, acc):
        c = lax.axis_index("c")
        s = lax.axis_index("s")
        wid = c * NS + s

        @pl.when(s == 0)
        def _():
            pltpu.sync_copy(zero_hbm, acc)

        def fill(i, carry):
            ones[i, :] = jnp.ones((H,), jnp.float32)
            return carry
        lax.fori_loop(0, CB, fill, 0)
        plsc.subcore_barrier()

        pltpu.sync_copy(dst_hbm.at[pl.ds(wid * nch, nch)], dstb)

        def step(j, carry):
            pltpu.sync_copy(ones, acc.at[dstb.at[j]], add=True)
            return carry
        lax.fori_loop(0, nch, step, 0)

        plsc.subcore_barrier()
        pltpu.sync_copy(acc.at[pl.ds(s * RPT, RPT)],
                        out_hbm.at[c, pl.ds(s * RPT, RPT)])

    return pl.kernel(
        body,
        out_type=jax.ShapeDtypeStruct((NC, NROW, H), jnp.float32),
        mesh=_sc_mesh(),
        scratch_types=[
            pltpu.VMEM((nch, CB), jnp.int32),
            pltpu.VMEM((CB, H), jnp.float32),
            pltpu.VMEM_SHARED((NROW, H), jnp.float32),
        ],
        compiler_params=pltpu.CompilerParams(use_tc_tiling_on_sc=False),
    )


def _make_gs_kernel(nch):
    def body(y_hbm, src_hbm, dst_hbm, zero_hbm, out_hbm,
             srcb, dstb, rows0, rows1, acc, sem0, sem1):
        c = lax.axis_index("c")
        s = lax.axis_index("s")
        wid = c * NS + s

        @pl.when(s == 0)
        def _():
            pltpu.sync_copy(zero_hbm, acc)

        pltpu.sync_copy(src_hbm.at[pl.ds(wid * nch, nch)], srcb)
        pltpu.sync_copy(dst_hbm.at[pl.ds(wid * nch, nch)], dstb)
        plsc.subcore_barrier()

        def step(jj, carry):
            j0 = jj * 2
            cp_a = pltpu.async_copy(y_hbm.at[srcb.at[j0]], rows0, sem0)
            cp_b = pltpu.async_copy(y_hbm.at[srcb.at[j0 + 1]], rows1, sem1)
            cp_a.wait()
            pltpu.sync_copy(rows0, acc.at[dstb.at[j0]], add=True)
            cp_b.wait()
            pltpu.sync_copy(rows1, acc.at[dstb.at[j0 + 1]], add=True)
            return carry
        lax.fori_loop(0, nch // 2, step, 0)

        plsc.subcore_barrier()
        pltpu.sync_copy(acc.at[pl.ds(s * RPT, RPT)],
                        out_hbm.at[c, pl.ds(s * RPT, RPT)])

    return pl.kernel(
        body,
        out_type=jax.ShapeDtypeStruct((NC, NROW, H), jnp.float32),
        mesh=_sc_mesh(),
        scratch_types=[
            pltpu.VMEM((nch, CB), jnp.int32),
            pltpu.VMEM((nch, CB), jnp.int32),
            pltpu.VMEM((CB, H), jnp.float32),
            pltpu.VMEM((CB, H), jnp.float32),
            pltpu.VMEM_SHARED((NROW, H), jnp.float32),
            pltpu.SemaphoreType.DMA,
            pltpu.SemaphoreType.DMA,
        ],
        compiler_params=pltpu.CompilerParams(use_tc_tiling_on_sc=False),
    )


def _tc1_body(dacc_ref, x_ref, w1_ref, dinv_ref, y1_ref):
    deg = dacc_ref[0, :N, :] + dacc_ref[1, :N, :] + 1.0
    dinv = lax.rsqrt(deg)
    h = jnp.dot(x_ref[...], w1_ref[...], preferred_element_type=jnp.float32)
    dinv_ref[...] = dinv
    y1_ref[...] = dinv * h


def _tc2_body(aacc_ref, y1_ref, dinv_ref, b1_ref, y2_ref):
    dinv = dinv_ref[...]
    z = dinv * (aacc_ref[0, :N, :] + aacc_ref[1, :N, :] + y1_ref[...]) + b1_ref[...]
    y2_ref[...] = dinv * jnp.maximum(z, 0.0)


def _tc3_body(aacc_ref, y2_ref, dinv_ref, w2_ref, b2_ref, out_ref):
    t = dinv_ref[...] * (aacc_ref[0, :N, :] + aacc_ref[1, :N, :] + y2_ref[...])
    out_ref[...] = (jnp.dot(t, w2_ref[...], preferred_element_type=jnp.float32)
                    + b2_ref[...])


@functools.lru_cache(maxsize=4)
def _build(e_total):
    nch = -(-(-(-e_total // (NW * CB))) // 8) * 8
    epw = nch * CB
    deg_kernel = _make_deg_kernel(nch)
    gs_kernel = _make_gs_kernel(nch)

    tc1 = pl.pallas_call(
        _tc1_body,
        out_shape=[jax.ShapeDtypeStruct((N, H), jnp.float32)] * 2,
    )
    tc2 = pl.pallas_call(
        _tc2_body,
        out_shape=jax.ShapeDtypeStruct((N, H), jnp.float32),
    )
    tc3 = pl.pallas_call(
        _tc3_body,
        out_shape=jax.ShapeDtypeStruct((N, C), jnp.float32),
    )

    @jax.jit
    def run(x, src2, dst2, w1, b1r, w2, b2r):
        zero_acc = jnp.zeros((NROW, H), jnp.float32)
        dacc = deg_kernel(dst2, zero_acc)
        dinv, y1 = tc1(dacc, x, w1)
        aacc1 = gs_kernel(y1, src2, dst2, zero_acc)
        y2 = tc2(aacc1, y1, dinv, b1r)
        aacc2 = gs_kernel(y2, src2, dst2, zero_acc)
        return tc3(aacc2, y2, dinv, w2, b2r)

    return run


def kernel(x, edge_index, W1, b1, W2, b2):
    src = edge_index[0]
    dst = edge_index[1]
    e_total = src.shape[0]
    nch = -(-(-(-e_total // (NW * CB))) // 8) * 8
    pad = NW * nch * CB - e_total
    src_p = jnp.concatenate([src, jnp.zeros((pad,), src.dtype)])
    dst_p = jnp.concatenate([dst, jnp.full((pad,), N, dst.dtype)])
    src2 = src_p.reshape(-1, CB)
    dst2 = dst_p.reshape(-1, CB)
    return _build(e_total)(x, src2, dst2, W1,
                           b1.reshape(1, H), W2, b2.reshape(1, C))


# speedup vs baseline: 57.0955x; 1.8256x over previous
import functools

import jax
import jax.numpy as jnp
from jax import lax
from jax.experimental import pallas as pl
from jax.experimental.pallas import tpu as pltpu
from jax.experimental.pallas import tpu_sc as plsc

N = 10000
D = 128
H = 16
C = 2
NC = 2
NS = 16
NW = NC * NS
CB = 128
BR = 8
NROW = N + 112
RPT = NROW // NS


def _sc_mesh():
    return plsc.VectorSubcoreMesh(core_axis_name="c", subcore_axis_name="s")


def _make_deg_kernel(nch):
    def body(dst_hbm, zero_hbm, out_hbm, dstb, ones, acc):
        c = lax.axis_index("c")
        s = lax.axis_index("s")
        wid = c * NS + s

        @pl.when(s == 0)
        def _():
            pltpu.sync_copy(zero_hbm, acc)

        def fill(i, carry):
            ones[i, :] = jnp.ones((H,), jnp.float32)
            return carry
        lax.fori_loop(0, CB, fill, 0)
        plsc.subcore_barrier()

        epw = nch * CB
        pltpu.sync_copy(dst_hbm.at[pl.ds(wid * epw, epw)], dstb)

        def step(j, carry):
            pltpu.sync_copy(ones, acc.at[dstb.at[pl.ds(j * CB, CB)]], add=True)
            return carry
        lax.fori_loop(0, nch, step, 0)

        plsc.subcore_barrier()
        pltpu.sync_copy(acc.at[pl.ds(s * RPT, RPT)],
                        out_hbm.at[c, pl.ds(s * RPT, RPT)])

    return pl.kernel(
        body,
        out_type=jax.ShapeDtypeStruct((NC, NROW, H), jnp.float32),
        mesh=_sc_mesh(),
        scratch_types=[
            pltpu.VMEM((nch * CB,), jnp.int32),
            pltpu.VMEM((CB, H), jnp.float32),
            pltpu.VMEM_SHARED((NROW, H), jnp.float32),
        ],
        compiler_params=pltpu.CompilerParams(use_tc_tiling_on_sc=False),
    )


def _make_gs_kernel(nch):
    def body(y_hbm, src_hbm, dst_hbm, zero_hbm, out_hbm,
             srcb, dstb, rows0, rows1, acc, sem0, sem1):
        c = lax.axis_index("c")
        s = lax.axis_index("s")
        wid = c * NS + s

        @pl.when(s == 0)
        def _():
            pltpu.sync_copy(zero_hbm, acc)

        epw = nch * CB
        pltpu.sync_copy(src_hbm.at[pl.ds(wid * epw, epw)], srcb)
        pltpu.sync_copy(dst_hbm.at[pl.ds(wid * epw, epw)], dstb)
        plsc.subcore_barrier()

        eb = BR * CB
        def step(jj, carry):
            a = jj * 2 * eb
            b = a + eb
            cp_a = pltpu.async_copy(y_hbm.at[srcb.at[pl.ds(a, eb)]], rows0, sem0)
            cp_b = pltpu.async_copy(y_hbm.at[srcb.at[pl.ds(b, eb)]], rows1, sem1)
            cp_a.wait()
            pltpu.sync_copy(rows0, acc.at[dstb.at[pl.ds(a, eb)]], add=True)
            cp_b.wait()
            pltpu.sync_copy(rows1, acc.at[dstb.at[pl.ds(b, eb)]], add=True)
            return carry
        lax.fori_loop(0, nch // (2 * BR), step, 0)

        plsc.subcore_barrier()
        pltpu.sync_copy(acc.at[pl.ds(s * RPT, RPT)],
                        out_hbm.at[c, pl.ds(s * RPT, RPT)])

    return pl.kernel(
        body,
        out_type=jax.ShapeDtypeStruct((NC, NROW, H), jnp.float32),
        mesh=_sc_mesh(),
        scratch_types=[
            pltpu.VMEM((nch * CB,), jnp.int32),
            pltpu.VMEM((nch * CB,), jnp.int32),
            pltpu.VMEM((BR * CB, H), jnp.float32),
            pltpu.VMEM((BR * CB, H), jnp.float32),
            pltpu.VMEM_SHARED((NROW, H), jnp.float32),
            pltpu.SemaphoreType.DMA,
            pltpu.SemaphoreType.DMA,
        ],
        compiler_params=pltpu.CompilerParams(use_tc_tiling_on_sc=False),
    )


def _tc1_body(dacc_ref, x_ref, w1_ref, dinv_ref, y1_ref):
    deg = dacc_ref[0, :N, :] + dacc_ref[1, :N, :] + 1.0
    dinv = lax.rsqrt(deg)
    h = jnp.dot(x_ref[...], w1_ref[...], preferred_element_type=jnp.float32)
    dinv_ref[...] = dinv
    y1_ref[...] = dinv * h


def _tc2_body(aacc_ref, y1_ref, dinv_ref, b1_ref, y2_ref):
    dinv = dinv_ref[...]
    z = dinv * (aacc_ref[0, :N, :] + aacc_ref[1, :N, :] + y1_ref[...]) + b1_ref[...]
    y2_ref[...] = dinv * jnp.maximum(z, 0.0)


def _tc3_body(aacc_ref, y2_ref, dinv_ref, w2_ref, b2_ref, out_ref):
    t = dinv_ref[...] * (aacc_ref[0, :N, :] + aacc_ref[1, :N, :] + y2_ref[...])
    out_ref[...] = (jnp.dot(t, w2_ref[...], preferred_element_type=jnp.float32)
                    + b2_ref[...])


@functools.lru_cache(maxsize=4)
def _build(e_total):
    nch = -(-(-(-e_total // (NW * CB))) // 8) * 8
    epw = nch * CB
    deg_kernel = _make_deg_kernel(nch)
    gs_kernel = _make_gs_kernel(nch)

    tc1 = pl.pallas_call(
        _tc1_body,
        out_shape=[jax.ShapeDtypeStruct((N, H), jnp.float32)] * 2,
    )
    tc2 = pl.pallas_call(
        _tc2_body,
        out_shape=jax.ShapeDtypeStruct((N, H), jnp.float32),
    )
    tc3 = pl.pallas_call(
        _tc3_body,
        out_shape=jax.ShapeDtypeStruct((N, C), jnp.float32),
    )

    @jax.jit
    def run(x, src2, dst2, w1, b1r, w2, b2r):
        zero_acc = jnp.zeros((NROW, H), jnp.float32)
        dacc = deg_kernel(dst2, zero_acc)
        dinv, y1 = tc1(dacc, x, w1)
        aacc1 = gs_kernel(y1, src2, dst2, zero_acc)
        y2 = tc2(aacc1, y1, dinv, b1r)
        aacc2 = gs_kernel(y2, src2, dst2, zero_acc)
        return tc3(aacc2, y2, dinv, w2, b2r)

    return run


def kernel(x, edge_index, W1, b1, W2, b2):
    src = edge_index[0]
    dst = edge_index[1]
    e_total = src.shape[0]
    nch = -(-(-(-e_total // (NW * CB))) // 8) * 8
    pad = NW * nch * CB - e_total
    pad_ids = jnp.arange(pad, dtype=src.dtype)
    src_p = jnp.concatenate([src, pad_ids % N])
    dst_p = jnp.concatenate([dst, N + pad_ids % (NROW - N)])
    return _build(e_total)(x, src_p, dst_p, W1,
                           b1.reshape(1, H), W2, b2.reshape(1, C))


# speedup vs baseline: 64.5955x; 1.1314x over previous
import functools

import jax
import jax.numpy as jnp
from jax import lax
from jax.experimental import pallas as pl
from jax.experimental.pallas import tpu as pltpu
from jax.experimental.pallas import tpu_sc as plsc

N = 10000
D = 128
H = 16
C = 2
NC = 2
NS = 16
NW = NC * NS
CB = 128
BR = 8
NROW = N + 112
RPT = NROW // NS


def _sc_mesh():
    return plsc.VectorSubcoreMesh(core_axis_name="c", subcore_axis_name="s")


def _rsqrt16(x):
    i = lax.bitcast_convert_type(x, jnp.int32)
    i = jnp.int32(0x5F3759DF) - lax.shift_right_logical(i, 1)
    y = lax.bitcast_convert_type(i, jnp.float32)
    y = y * (1.5 - 0.5 * x * y * y)
    y = y * (1.5 - 0.5 * x * y * y)
    y = y * (1.5 - 0.5 * x * y * y)
    return y


def _make_deg_kernel(nch):
    def body(dst_hbm, zero_hbm, out_hbm, dstb, ones, acc):
        c = lax.axis_index("c")
        s = lax.axis_index("s")
        wid = c * NS + s

        @pl.when(s == 0)
        def _():
            pltpu.sync_copy(zero_hbm, acc)

        def fill(i, carry):
            ones[i, :] = jnp.ones((H,), jnp.float32)
            return carry
        lax.fori_loop(0, CB, fill, 0)
        plsc.subcore_barrier()

        epw = nch * CB
        pltpu.sync_copy(dst_hbm.at[pl.ds(wid * epw, epw)], dstb)

        def step(j, carry):
            pltpu.sync_copy(ones, acc.at[dstb.at[pl.ds(j * CB, CB)]], add=True)
            return carry
        lax.fori_loop(0, nch, step, 0)

        plsc.subcore_barrier()
        pltpu.sync_copy(acc.at[pl.ds(s * RPT, RPT)],
                        out_hbm.at[c, pl.ds(s * RPT, RPT)])

    return pl.kernel(
        body,
        out_type=jax.ShapeDtypeStruct((NC, NROW, H), jnp.float32),
        mesh=_sc_mesh(),
        scratch_types=[
            pltpu.VMEM((nch * CB,), jnp.int32),
            pltpu.VMEM((CB, H), jnp.float32),
            pltpu.VMEM_SHARED((NROW, H), jnp.float32),
        ],
        compiler_params=pltpu.CompilerParams(use_tc_tiling_on_sc=False),
    )


def _edge_pass(nch, srcb, dstb, rows0, rows1, acc, ytab, sem0, sem1, wid):
    eb = BR * CB

    def step(jj, carry):
        a = jj * 2 * eb
        b = a + eb
        cp_a = pltpu.async_copy(ytab.at[srcb.at[pl.ds(a, eb)]], rows0, sem0)
        cp_b = pltpu.async_copy(ytab.at[srcb.at[pl.ds(b, eb)]], rows1, sem1)
        cp_a.wait()
        pltpu.sync_copy(rows0, acc.at[dstb.at[pl.ds(a, eb)]], add=True)
        cp_b.wait()
        pltpu.sync_copy(rows1, acc.at[dstb.at[pl.ds(b, eb)]], add=True)
        return carry
    lax.fori_loop(0, nch // (2 * BR), step, 0)


def _make_gs1_kernel(nch):
    def body(h_hbm, dacc_hbm, src_hbm, dst_hbm, zero_hbm,
             aacc_hbm, dinv_hbm,
             srcb, dstb, rows0, rows1, t0, t1, th, acc, ytab, sem0, sem1):
        c = lax.axis_index("c")
        s = lax.axis_index("s")
        wid = c * NS + s
        base = s * RPT

        @pl.when(s == 0)
        def _():
            pltpu.sync_copy(zero_hbm, acc)

        epw = nch * CB
        pltpu.sync_copy(src_hbm.at[pl.ds(wid * epw, epw)], srcb)
        pltpu.sync_copy(dst_hbm.at[pl.ds(wid * epw, epw)], dstb)

        pltpu.sync_copy(dacc_hbm.at[0, pl.ds(base, RPT)], t0)
        pltpu.sync_copy(dacc_hbm.at[1, pl.ds(base, RPT)], t1)
        pltpu.sync_copy(h_hbm.at[pl.ds(base, RPT)], th)

        def row(i, carry):
            deg = t0[i, :] + t1[i, :] + 1.0
            dv = _rsqrt16(deg)
            t0[i, :] = dv
            th[i, :] = dv * th[i, :]
            return carry
        lax.fori_loop(0, RPT, row, 0)

        pltpu.sync_copy(th, ytab.at[pl.ds(base, RPT)])

        @pl.when(c == 0)
        def _():
            pltpu.sync_copy(t0, dinv_hbm.at[pl.ds(base, RPT)])

        plsc.subcore_barrier()
        _edge_pass(nch, srcb, dstb, rows0, rows1, acc, ytab, sem0, sem1, wid)
        plsc.subcore_barrier()
        pltpu.sync_copy(acc.at[pl.ds(base, RPT)],
                        aacc_hbm.at[c, pl.ds(base, RPT)])

    return pl.kernel(
        body,
        out_type=(jax.ShapeDtypeStruct((NC, NROW, H), jnp.float32),
                  jax.ShapeDtypeStruct((NROW, H), jnp.float32)),
        mesh=_sc_mesh(),
        scratch_types=[
            pltpu.VMEM((nch * CB,), jnp.int32),
            pltpu.VMEM((nch * CB,), jnp.int32),
            pltpu.VMEM((BR * CB, H), jnp.float32),
            pltpu.VMEM((BR * CB, H), jnp.float32),
            pltpu.VMEM((RPT, H), jnp.float32),
            pltpu.VMEM((RPT, H), jnp.float32),
            pltpu.VMEM((RPT, H), jnp.float32),
            pltpu.VMEM_SHARED((NROW, H), jnp.float32),
            pltpu.VMEM_SHARED((NROW, H), jnp.float32),
            pltpu.SemaphoreType.DMA,
            pltpu.SemaphoreType.DMA,
        ],
        compiler_params=pltpu.CompilerParams(use_tc_tiling_on_sc=False),
    )


def _make_gs2_kernel(nch):
    def body(h_hbm, dinv_hbm, aacc_hbm, src_hbm, dst_hbm, b1_hbm, zero_hbm,
             aacc2_hbm, y2_hbm,
             srcb, dstb, rows0, rows1, t0, t1, th, tdv, tb, acc, ytab,
             sem0, sem1):
        c = lax.axis_index("c")
        s = lax.axis_index("s")
        wid = c * NS + s
        base = s * RPT

        @pl.when(s == 0)
        def _():
            pltpu.sync_copy(zero_hbm, acc)

        epw = nch * CB
        pltpu.sync_copy(src_hbm.at[pl.ds(wid * epw, epw)], srcb)
        pltpu.sync_copy(dst_hbm.at[pl.ds(wid * epw, epw)], dstb)

        pltpu.sync_copy(aacc_hbm.at[0, pl.ds(base, RPT)], t0)
        pltpu.sync_copy(aacc_hbm.at[1, pl.ds(base, RPT)], t1)
        pltpu.sync_copy(h_hbm.at[pl.ds(base, RPT)], th)
        pltpu.sync_copy(dinv_hbm.at[pl.ds(base, RPT)], tdv)
        pltpu.sync_copy(b1_hbm, tb)
        b1v = tb[...]

        def row(i, carry):
            dv = tdv[i, :]
            y1 = dv * th[i, :]
            z = dv * (t0[i, :] + t1[i, :] + y1) + b1v
            th[i, :] = dv * jnp.maximum(z, 0.0)
            return carry
        lax.fori_loop(0, RPT, row, 0)

        pltpu.sync_copy(th, ytab.at[pl.ds(base, RPT)])

        @pl.when(c == 0)
        def _():
            pltpu.sync_copy(th, y2_hbm.at[pl.ds(base, RPT)])

        plsc.subcore_barrier()
        _edge_pass(nch, srcb, dstb, rows0, rows1, acc, ytab, sem0, sem1, wid)
        plsc.subcore_barrier()
        pltpu.sync_copy(acc.at[pl.ds(base, RPT)],
                        aacc2_hbm.at[c, pl.ds(base, RPT)])

    return pl.kernel(
        body,
        out_type=(jax.ShapeDtypeStruct((NC, NROW, H), jnp.float32),
                  jax.ShapeDtypeStruct((NROW, H), jnp.float32)),
        mesh=_sc_mesh(),
        scratch_types=[
            pltpu.VMEM((nch * CB,), jnp.int32),
            pltpu.VMEM((nch * CB,), jnp.int32),
            pltpu.VMEM((BR * CB, H), jnp.float32),
            pltpu.VMEM((BR * CB, H), jnp.float32),
            pltpu.VMEM((RPT, H), jnp.float32),
            pltpu.VMEM((RPT, H), jnp.float32),
            pltpu.VMEM((RPT, H), jnp.float32),
            pltpu.VMEM((RPT, H), jnp.float32),
            pltpu.VMEM((H,), jnp.float32),
            pltpu.VMEM_SHARED((NROW, H), jnp.float32),
            pltpu.VMEM_SHARED((NROW, H), jnp.float32),
            pltpu.SemaphoreType.DMA,
            pltpu.SemaphoreType.DMA,
        ],
        compiler_params=pltpu.CompilerParams(use_tc_tiling_on_sc=False),
    )


def _tch_body(x_ref, w1_ref, h_ref):
    h_ref[:N, :] = jnp.dot(x_ref[...], w1_ref[...],
                           preferred_element_type=jnp.float32)
    h_ref[N:, :] = jnp.zeros((NROW - N, H), jnp.float32)


def _tc3_body(aacc_ref, y2_ref, dinv_ref, w2_ref, b2_ref, out_ref):
    t = dinv_ref[:N, :] * (aacc_ref[0, :N, :] + aacc_ref[1, :N, :]
                           + y2_ref[:N, :])
    out_ref[...] = (jnp.dot(t, w2_ref[...], preferred_element_type=jnp.float32)
                    + b2_ref[...])


@functools.lru_cache(maxsize=4)
def _build(e_total):
    nch = -(-(-(-e_total // (NW * CB))) // (2 * BR)) * (2 * BR)
    deg_kernel = _make_deg_kernel(nch)
    gs1_kernel = _make_gs1_kernel(nch)
    gs2_kernel = _make_gs2_kernel(nch)

    tch = pl.pallas_call(
        _tch_body,
        out_shape=jax.ShapeDtypeStruct((NROW, H), jnp.float32),
    )
    tc3 = pl.pallas_call(
        _tc3_body,
        out_shape=jax.ShapeDtypeStruct((N, C), jnp.float32),
    )

    @jax.jit
    def run(x, src_p, dst_p, w1, b1, w2, b2r):
        zero_acc = jnp.zeros((NROW, H), jnp.float32)
        h = tch(x, w1)
        dacc = deg_kernel(dst_p, zero_acc)
        aacc1, dinv = gs1_kernel(h, dacc, src_p, dst_p, zero_acc)
        aacc2, y2 = gs2_kernel(h, dinv, aacc1, src_p, dst_p, b1, zero_acc)
        return tc3(aacc2, y2, dinv, w2, b2r)

    return run


def kernel(x, edge_index, W1, b1, W2, b2):
    src = edge_index[0]
    dst = edge_index[1]
    e_total = src.shape[0]
    nch = -(-(-(-e_total // (NW * CB))) // (2 * BR)) * (2 * BR)
    pad = NW * nch * CB - e_total
    pad_ids = jnp.arange(pad, dtype=src.dtype)
    src_p = jnp.concatenate([src, pad_ids % N])
    dst_p = jnp.concatenate([dst, N + pad_ids % (NROW - N)])
    return _build(e_total)(x, src_p, dst_p, W1, b1, W2, b2.reshape(1, C))


# speedup vs baseline: 76.8532x; 1.1898x over previous
import functools

import jax
import jax.numpy as jnp
from jax import lax
from jax.experimental import pallas as pl
from jax.experimental.pallas import tpu as pltpu
from jax.experimental.pallas import tpu_sc as plsc

N = 10000
D = 128
H = 16
C = 2
NC = 2
NS = 16
NW = NC * NS
CB = 128
EB = 512
NBUF = 4
LAG = 2
NROW = N + 112
RPT = NROW // NS


def _sc_mesh():
    return plsc.VectorSubcoreMesh(core_axis_name="c", subcore_axis_name="s")


def _rsqrt16(x):
    i = lax.bitcast_convert_type(x, jnp.int32)
    i = jnp.int32(0x5F3759DF) - lax.shift_right_logical(i, 1)
    y = lax.bitcast_convert_type(i, jnp.float32)
    y = y * (1.5 - 0.5 * x * y * y)
    y = y * (1.5 - 0.5 * x * y * y)
    y = y * (1.5 - 0.5 * x * y * y)
    return y


def _make_deg_kernel(nch):
    def body(dst_hbm, zero_hbm, out_hbm, dstb, ones, acc, sem):
        c = lax.axis_index("c")
        s = lax.axis_index("s")
        wid = c * NS + s

        @pl.when(s == 0)
        def _():
            pltpu.sync_copy(zero_hbm, acc)

        @plsc.parallel_loop(0, EB, 1, unroll=8)
        def _(i):
            ones[i, :] = jnp.ones((H,), jnp.float32)

        epw = nch * CB
        pltpu.sync_copy(dst_hbm.at[pl.ds(wid * epw, epw)], dstb)
        plsc.subcore_barrier()

        descs = [
            pltpu.async_copy(ones, acc.at[dstb.at[pl.ds(j * EB, EB)]], sem,
                             add=True)
            for j in range(nch * CB // EB)
        ]
        for d in descs:
            d.wait()

        plsc.subcore_barrier()
        pltpu.sync_copy(acc.at[pl.ds(s * RPT, RPT)],
                        out_hbm.at[c, pl.ds(s * RPT, RPT)])

    return pl.kernel(
        body,
        out_type=jax.ShapeDtypeStruct((NC, NROW, H), jnp.float32),
        mesh=_sc_mesh(),
        scratch_types=[
            pltpu.VMEM((nch * CB,), jnp.int32),
            pltpu.VMEM((EB, H), jnp.float32),
            pltpu.VMEM_SHARED((NROW, H), jnp.float32),
            pltpu.SemaphoreType.DMA,
        ],
        compiler_params=pltpu.CompilerParams(use_tc_tiling_on_sc=False),
    )


def _edge_pass(nsc, srcb, dstb, rows, gsems, ssems, acc, ytab):
    gat = [None] * NBUF
    sca = [None] * NBUF
    for j in range(nsc + LAG):
        if j < nsc:
            b = j % NBUF
            if sca[b] is not None:
                sca[b].wait()
            gat[b] = pltpu.async_copy(
                ytab.at[srcb.at[pl.ds(j * EB, EB)]], rows[b], gsems[b])
        i = j - LAG
        if i >= 0:
            bi = i % NBUF
            gat[bi].wait()
            sca[bi] = pltpu.async_copy(
                rows[bi], acc.at[dstb.at[pl.ds(i * EB, EB)]], ssems[bi],
                add=True)
    for i in range(max(0, nsc - NBUF), nsc):
        sca[i % NBUF].wait()


def _make_gs1_kernel(nch):
    def body(h_hbm, dacc_hbm, src_hbm, dst_hbm, zero_hbm,
             aacc_hbm, dinv_hbm,
             srcb, dstb, r0, r1, r2, r3, t0, t1, th, acc, ytab,
             g0, g1, g2, g3, s0, s1, s2, s3):
        c = lax.axis_index("c")
        s = lax.axis_index("s")
        wid = c * NS + s
        base = s * RPT

        @pl.when(s == 0)
        def _():
            pltpu.sync_copy(zero_hbm, acc)

        epw = nch * CB
        pltpu.sync_copy(src_hbm.at[pl.ds(wid * epw, epw)], srcb)
        pltpu.sync_copy(dst_hbm.at[pl.ds(wid * epw, epw)], dstb)

        pltpu.sync_copy(dacc_hbm.at[0, pl.ds(base, RPT)], t0)
        pltpu.sync_copy(dacc_hbm.at[1, pl.ds(base, RPT)], t1)
        pltpu.sync_copy(h_hbm.at[pl.ds(base, RPT)], th)

        @plsc.parallel_loop(0, RPT, 1, unroll=8)
        def _(i):
            deg = t0[i, :] + t1[i, :] + 1.0
            dv = _rsqrt16(deg)
            t0[i, :] = dv
            th[i, :] = dv * th[i, :]

        pltpu.sync_copy(th, ytab.at[pl.ds(base, RPT)])

        @pl.when(c == 0)
        def _():
            pltpu.sync_copy(t0, dinv_hbm.at[pl.ds(base, RPT)])

        plsc.subcore_barrier()
        _edge_pass(nch * CB // EB, srcb, dstb, [r0, r1, r2, r3],
                   [g0, g1, g2, g3], [s0, s1, s2, s3], acc, ytab)
        plsc.subcore_barrier()
        pltpu.sync_copy(acc.at[pl.ds(base, RPT)],
                        aacc_hbm.at[c, pl.ds(base, RPT)])

    return pl.kernel(
        body,
        out_type=(jax.ShapeDtypeStruct((NC, NROW, H), jnp.float32),
                  jax.ShapeDtypeStruct((NROW, H), jnp.float32)),
        mesh=_sc_mesh(),
        scratch_types=[
            pltpu.VMEM((nch * CB,), jnp.int32),
            pltpu.VMEM((nch * CB,), jnp.int32),
            pltpu.VMEM((EB, H), jnp.float32),
            pltpu.VMEM((EB, H), jnp.float32),
            pltpu.VMEM((EB, H), jnp.float32),
            pltpu.VMEM((EB, H), jnp.float32),
            pltpu.VMEM((RPT, H), jnp.float32),
            pltpu.VMEM((RPT, H), jnp.float32),
            pltpu.VMEM((RPT, H), jnp.float32),
            pltpu.VMEM_SHARED((NROW, H), jnp.float32),
            pltpu.VMEM_SHARED((NROW, H), jnp.float32),
        ] + [pltpu.SemaphoreType.DMA] * 8,
        compiler_params=pltpu.CompilerParams(use_tc_tiling_on_sc=False),
    )


def _make_gs2_kernel(nch):
    def body(h_hbm, dinv_hbm, aacc_hbm, src_hbm, dst_hbm, b1_hbm, zero_hbm,
             aacc2_hbm, y2_hbm,
             srcb, dstb, r0, r1, r2, r3, t0, t1, th, tdv, tb, acc, ytab,
             g0, g1, g2, g3, s0, s1, s2, s3):
        c = lax.axis_index("c")
        s = lax.axis_index("s")
        wid = c * NS + s
        base = s * RPT

        @pl.when(s == 0)
        def _():
            pltpu.sync_copy(zero_hbm, acc)

        epw = nch * CB
        pltpu.sync_copy(src_hbm.at[pl.ds(wid * epw, epw)], srcb)
        pltpu.sync_copy(dst_hbm.at[pl.ds(wid * epw, epw)], dstb)

        pltpu.sync_copy(aacc_hbm.at[0, pl.ds(base, RPT)], t0)
        pltpu.sync_copy(aacc_hbm.at[1, pl.ds(base, RPT)], t1)
        pltpu.sync_copy(h_hbm.at[pl.ds(base, RPT)], th)
        pltpu.sync_copy(dinv_hbm.at[pl.ds(base, RPT)], tdv)
        pltpu.sync_copy(b1_hbm, tb)
        b1v = tb[...]

        @plsc.parallel_loop(0, RPT, 1, unroll=8)
        def _(i):
            dv = tdv[i, :]
            y1 = dv * th[i, :]
            z = dv * (t0[i, :] + t1[i, :] + y1) + b1v
            th[i, :] = dv * jnp.maximum(z, 0.0)

        pltpu.sync_copy(th, ytab.at[pl.ds(base, RPT)])

        @pl.when(c == 0)
        def _():
            pltpu.sync_copy(th, y2_hbm.at[pl.ds(base, RPT)])

        plsc.subcore_barrier()
        _edge_pass(nch * CB // EB, srcb, dstb, [r0, r1, r2, r3],
                   [g0, g1, g2, g3], [s0, s1, s2, s3], acc, ytab)
        plsc.subcore_barrier()
        pltpu.sync_copy(acc.at[pl.ds(base, RPT)],
                        aacc2_hbm.at[c, pl.ds(base, RPT)])

    return pl.kernel(
        body,
        out_type=(jax.ShapeDtypeStruct((NC, NROW, H), jnp.float32),
                  jax.ShapeDtypeStruct((NROW, H), jnp.float32)),
        mesh=_sc_mesh(),
        scratch_types=[
            pltpu.VMEM((nch * CB,), jnp.int32),
            pltpu.VMEM((nch * CB,), jnp.int32),
            pltpu.VMEM((EB, H), jnp.float32),
            pltpu.VMEM((EB, H), jnp.float32),
            pltpu.VMEM((EB, H), jnp.float32),
            pltpu.VMEM((EB, H), jnp.float32),
            pltpu.VMEM((RPT, H), jnp.float32),
            pltpu.VMEM((RPT, H), jnp.float32),
            pltpu.VMEM((RPT, H), jnp.float32),
            pltpu.VMEM((RPT, H), jnp.float32),
            pltpu.VMEM((H,), jnp.float32),
            pltpu.VMEM_SHARED((NROW, H), jnp.float32),
            pltpu.VMEM_SHARED((NROW, H), jnp.float32),
        ] + [pltpu.SemaphoreType.DMA] * 8,
        compiler_params=pltpu.CompilerParams(use_tc_tiling_on_sc=False),
    )


def _tch_body(x_ref, w1_ref, h_ref):
    h_ref[:N, :] = jnp.dot(x_ref[...], w1_ref[...],
                           preferred_element_type=jnp.float32)
    h_ref[N:, :] = jnp.zeros((NROW - N, H), jnp.float32)


def _tc3_body(aacc_ref, y2_ref, dinv_ref, w2_ref, b2_ref, out_ref):
    t = dinv_ref[:N, :] * (aacc_ref[0, :N, :] + aacc_ref[1, :N, :]
                           + y2_ref[:N, :])
    out_ref[...] = (jnp.dot(t, w2_ref[...], preferred_element_type=jnp.float32)
                    + b2_ref[...])


@functools.lru_cache(maxsize=4)
def _build(e_total):
    nch = -(-(-(-e_total // (NW * CB))) // 16) * 16
    deg_kernel = _make_deg_kernel(nch)
    gs1_kernel = _make_gs1_kernel(nch)
    gs2_kernel = _make_gs2_kernel(nch)

    tch = pl.pallas_call(
        _tch_body,
        out_shape=jax.ShapeDtypeStruct((NROW, H), jnp.float32),
    )
    tc3 = pl.pallas_call(
        _tc3_body,
        out_shape=jax.ShapeDtypeStruct((N, C), jnp.float32),
    )

    @jax.jit
    def run(x, src_p, dst_p, w1, b1, w2, b2r):
        zero_acc = jnp.zeros((NROW, H), jnp.float32)
        h = tch(x, w1)
        dacc = deg_kernel(dst_p, zero_acc)
        aacc1, dinv = gs1_kernel(h, dacc, src_p, dst_p, zero_acc)
        aacc2, y2 = gs2_kernel(h, dinv, aacc1, src_p, dst_p, b1, zero_acc)
        return tc3(aacc2, y2, dinv, w2, b2r)

    return run


def kernel(x, edge_index, W1, b1, W2, b2):
    src = edge_index[0]
    dst = edge_index[1]
    e_total = src.shape[0]
    nch = -(-(-(-e_total // (NW * CB))) // 16) * 16
    pad = NW * nch * CB - e_total
    pad_ids = jnp.arange(pad, dtype=src.dtype)
    src_p = jnp.concatenate([src, pad_ids % N])
    dst_p = jnp.concatenate([dst, N + pad_ids % (NROW - N)])
    return _build(e_total)(x, src_p, dst_p, W1, b1, W2, b2.reshape(1, C))


# speedup vs baseline: 86.4326x; 1.1246x over previous
import functools

import jax
import jax.numpy as jnp
from jax import lax
from jax.experimental import pallas as pl
from jax.experimental.pallas import tpu as pltpu
from jax.experimental.pallas import tpu_sc as plsc

N = 10000
D = 128
H = 16
C = 2
NC = 2
NS = 16
NW = NC * NS
CB = 128
EB = 512
NBUF = 4
LAG = 2
NROW = N + 112
RPT = NROW // NS


def _sc_mesh():
    return plsc.VectorSubcoreMesh(core_axis_name="c", subcore_axis_name="s")


def _rsqrt16(x):
    i = lax.bitcast_convert_type(x, jnp.int32)
    i = jnp.int32(0x5F3759DF) - lax.shift_right_logical(i, 1)
    y = lax.bitcast_convert_type(i, jnp.float32)
    y = y * (1.5 - 0.5 * x * y * y)
    y = y * (1.5 - 0.5 * x * y * y)
    y = y * (1.5 - 0.5 * x * y * y)
    return y


def _make_deg_kernel(ew, rem):
    def body(dst_hbm, zero_hbm, out_hbm, dstb, ones, acc, sem):
        c = lax.axis_index("c")
        s = lax.axis_index("s")
        wid = c * NS + s

        @pl.when(s == 0)
        def _():
            pltpu.sync_copy(zero_hbm, acc)

        @plsc.parallel_loop(0, EB, 1, unroll=8)
        def _(i):
            ones[i, :] = jnp.ones((H,), jnp.float32)

        pltpu.sync_copy(dst_hbm.at[pl.ds(wid * ew, ew)], dstb.at[pl.ds(0, ew)])
        if rem:
            @pl.when(wid == NW - 1)
            def _():
                pltpu.sync_copy(dst_hbm.at[pl.ds(NW * ew, rem)],
                                dstb.at[pl.ds(ew, rem)])
        plsc.subcore_barrier()

        descs = [
            pltpu.async_copy(ones.at[pl.ds(0, ln)],
                             acc.at[dstb.at[pl.ds(off, ln)]], sem, add=True)
            for off, ln in _chunks(ew)
        ]
        for d in descs:
            d.wait()
        if rem:
            @pl.when(wid == NW - 1)
            def _():
                pltpu.sync_copy(ones.at[pl.ds(0, rem)],
                                acc.at[dstb.at[pl.ds(ew, rem)]], add=True)

        plsc.subcore_barrier()
        pltpu.sync_copy(acc.at[pl.ds(s * RPT, RPT)],
                        out_hbm.at[c, pl.ds(s * RPT, RPT)])

    return pl.kernel(
        body,
        out_type=jax.ShapeDtypeStruct((NC, NROW, H), jnp.float32),
        mesh=_sc_mesh(),
        scratch_types=[
            pltpu.VMEM((ew + EB,), jnp.int32),
            pltpu.VMEM((EB, H), jnp.float32),
            pltpu.VMEM_SHARED((NROW, H), jnp.float32),
            pltpu.SemaphoreType.DMA,
        ],
        compiler_params=pltpu.CompilerParams(use_tc_tiling_on_sc=False),
    )


def _chunks(ew):
    ck = [(j * EB, EB) for j in range(ew // EB)]
    if ew % EB:
        ck.append((ew - ew % EB, ew % EB))
    return ck


def _edge_pass(ew, srcb, dstb, rows, gsems, ssems, acc, ytab):
    ck = _chunks(ew)
    nsc = len(ck)
    gat = [None] * NBUF
    sca = [None] * NBUF
    for j in range(nsc + LAG):
        if j < nsc:
            b = j % NBUF
            if sca[b] is not None:
                sca[b].wait()
            off, ln = ck[j]
            gat[b] = pltpu.async_copy(
                ytab.at[srcb.at[pl.ds(off, ln)]],
                rows[b].at[pl.ds(0, ln)], gsems[b])
        i = j - LAG
        if i >= 0:
            bi = i % NBUF
            gat[bi].wait()
            off, ln = ck[i]
            sca[bi] = pltpu.async_copy(
                rows[bi].at[pl.ds(0, ln)], acc.at[dstb.at[pl.ds(off, ln)]],
                ssems[bi], add=True)
    for i in range(max(0, nsc - NBUF), nsc):
        sca[i % NBUF].wait()


def _load_idx(src_hbm, dst_hbm, srcb, dstb, wid, ew, rem):
    pltpu.sync_copy(src_hbm.at[pl.ds(wid * ew, ew)], srcb.at[pl.ds(0, ew)])
    pltpu.sync_copy(dst_hbm.at[pl.ds(wid * ew, ew)], dstb.at[pl.ds(0, ew)])
    if rem:
        @pl.when(wid == NW - 1)
        def _():
            pltpu.sync_copy(src_hbm.at[pl.ds(NW * ew, rem)],
                            srcb.at[pl.ds(ew, rem)])
            pltpu.sync_copy(dst_hbm.at[pl.ds(NW * ew, rem)],
                            dstb.at[pl.ds(ew, rem)])


def _rem_pass(srcb, dstb, row, sem, acc, ytab, wid, ew, rem):
    if rem:
        @pl.when(wid == NW - 1)
        def _():
            pltpu.async_copy(ytab.at[srcb.at[pl.ds(ew, rem)]],
                             row.at[pl.ds(0, rem)], sem).wait()
            pltpu.sync_copy(row.at[pl.ds(0, rem)],
                            acc.at[dstb.at[pl.ds(ew, rem)]], add=True)


def _make_gs1_kernel(ew, rem):
    def body(h_hbm, dacc_hbm, src_hbm, dst_hbm, zero_hbm,
             aacc_hbm, dinv_hbm,
             srcb, dstb, r0, r1, r2, r3, t0, t1, th, acc, ytab,
             g0, g1, g2, g3, s0, s1, s2, s3):
        c = lax.axis_index("c")
        s = lax.axis_index("s")
        wid = c * NS + s
        base = s * RPT

        @pl.when(s == 0)
        def _():
            pltpu.sync_copy(zero_hbm, acc)

        _load_idx(src_hbm, dst_hbm, srcb, dstb, wid, ew, rem)

        pltpu.sync_copy(dacc_hbm.at[0, pl.ds(base, RPT)], t0)
        pltpu.sync_copy(dacc_hbm.at[1, pl.ds(base, RPT)], t1)
        pltpu.sync_copy(h_hbm.at[pl.ds(base, RPT)], th)

        @plsc.parallel_loop(0, RPT, 1, unroll=8)
        def _(i):
            deg = t0[i, :] + t1[i, :] + 1.0
            dv = _rsqrt16(deg)
            t0[i, :] = dv
            th[i, :] = dv * th[i, :]

        pltpu.sync_copy(th, ytab.at[pl.ds(base, RPT)])

        @pl.when(c == 0)
        def _():
            pltpu.sync_copy(t0, dinv_hbm.at[pl.ds(base, RPT)])

        plsc.subcore_barrier()
        _edge_pass(ew, srcb, dstb, [r0, r1, r2, r3],
                   [g0, g1, g2, g3], [s0, s1, s2, s3], acc, ytab)
        _rem_pass(srcb, dstb, r0, g0, acc, ytab, wid, ew, rem)
        plsc.subcore_barrier()
        pltpu.sync_copy(acc.at[pl.ds(base, RPT)],
                        aacc_hbm.at[c, pl.ds(base, RPT)])

    return pl.kernel(
        body,
        out_type=(jax.ShapeDtypeStruct((NC, NROW, H), jnp.float32),
                  jax.ShapeDtypeStruct((NROW, H), jnp.float32)),
        mesh=_sc_mesh(),
        scratch_types=[
            pltpu.VMEM((ew + EB,), jnp.int32),
            pltpu.VMEM((ew + EB,), jnp.int32),
            pltpu.VMEM((EB, H), jnp.float32),
            pltpu.VMEM((EB, H), jnp.float32),
            pltpu.VMEM((EB, H), jnp.float32),
            pltpu.VMEM((EB, H), jnp.float32),
            pltpu.VMEM((RPT, H), jnp.float32),
            pltpu.VMEM((RPT, H), jnp.float32),
            pltpu.VMEM((RPT, H), jnp.float32),
            pltpu.VMEM_SHARED((NROW, H), jnp.float32),
            pltpu.VMEM_SHARED((NROW, H), jnp.float32),
        ] + [pltpu.SemaphoreType.DMA] * 8,
        compiler_params=pltpu.CompilerParams(use_tc_tiling_on_sc=False),
    )


def _make_gs2_kernel(ew, rem):
    def body(h_hbm, dinv_hbm, aacc_hbm, src_hbm, dst_hbm, b1_hbm, zero_hbm,
             aacc2_hbm, y2_hbm,
             srcb, dstb, r0, r1, r2, r3, t0, t1, th, tdv, tb, acc, ytab,
             g0, g1, g2, g3, s0, s1, s2, s3):
        c = lax.axis_index("c")
        s = lax.axis_index("s")
        wid = c * NS + s
        base = s * RPT

        @pl.when(s == 0)
        def _():
            pltpu.sync_copy(zero_hbm, acc)

        _load_idx(src_hbm, dst_hbm, srcb, dstb, wid, ew, rem)

        pltpu.sync_copy(aacc_hbm.at[0, pl.ds(base, RPT)], t0)
        pltpu.sync_copy(aacc_hbm.at[1, pl.ds(base, RPT)], t1)
        pltpu.sync_copy(h_hbm.at[pl.ds(base, RPT)], th)
        pltpu.sync_copy(dinv_hbm.at[pl.ds(base, RPT)], tdv)
        pltpu.sync_copy(b1_hbm, tb)
        b1v = tb[...]

        @plsc.parallel_loop(0, RPT, 1, unroll=8)
        def _(i):
            dv = tdv[i, :]
            y1 = dv * th[i, :]
            z = dv * (t0[i, :] + t1[i, :] + y1) + b1v
            th[i, :] = dv * jnp.maximum(z, 0.0)

        pltpu.sync_copy(th, ytab.at[pl.ds(base, RPT)])

        @pl.when(c == 0)
        def _():
            pltpu.sync_copy(th, y2_hbm.at[pl.ds(base, RPT)])

        plsc.subcore_barrier()
        _edge_pass(ew, srcb, dstb, [r0, r1, r2, r3],
                   [g0, g1, g2, g3], [s0, s1, s2, s3], acc, ytab)
        _rem_pass(srcb, dstb, r0, g0, acc, ytab, wid, ew, rem)
        plsc.subcore_barrier()
        pltpu.sync_copy(acc.at[pl.ds(base, RPT)],
                        aacc2_hbm.at[c, pl.ds(base, RPT)])

    return pl.kernel(
        body,
        out_type=(jax.ShapeDtypeStruct((NC, NROW, H), jnp.float32),
                  jax.ShapeDtypeStruct((NROW, H), jnp.float32)),
        mesh=_sc_mesh(),
        scratch_types=[
            pltpu.VMEM((ew + EB,), jnp.int32),
            pltpu.VMEM((ew + EB,), jnp.int32),
            pltpu.VMEM((EB, H), jnp.float32),
            pltpu.VMEM((EB, H), jnp.float32),
            pltpu.VMEM((EB, H), jnp.float32),
            pltpu.VMEM((EB, H), jnp.float32),
            pltpu.VMEM((RPT, H), jnp.float32),
            pltpu.VMEM((RPT, H), jnp.float32),
            pltpu.VMEM((RPT, H), jnp.float32),
            pltpu.VMEM((RPT, H), jnp.float32),
            pltpu.VMEM((H,), jnp.float32),
            pltpu.VMEM_SHARED((NROW, H), jnp.float32),
            pltpu.VMEM_SHARED((NROW, H), jnp.float32),
        ] + [pltpu.SemaphoreType.DMA] * 8,
        compiler_params=pltpu.CompilerParams(use_tc_tiling_on_sc=False),
    )


def _tch_body(x_ref, w1_ref, h_ref):
    h_ref[:N, :] = jnp.dot(x_ref[...], w1_ref[...],
                           preferred_element_type=jnp.float32)
    h_ref[N:, :] = jnp.zeros((NROW - N, H), jnp.float32)


def _tc3_body(aacc_ref, y2_ref, dinv_ref, w2p_ref, b2p_ref, out_ref):
    t = dinv_ref[...] * (aacc_ref[0] + aacc_ref[1] + y2_ref[...])
    out_ref[...] = (jnp.dot(t, w2p_ref[...],
                            preferred_element_type=jnp.float32) + b2p_ref[...])


@functools.lru_cache(maxsize=4)
def _build(e_total):
    ew = (e_total // NW) // 8 * 8
    rem = e_total - NW * ew
    deg_kernel = _make_deg_kernel(ew, rem)
    gs1_kernel = _make_gs1_kernel(ew, rem)
    gs2_kernel = _make_gs2_kernel(ew, rem)

    tch = pl.pallas_call(
        _tch_body,
        out_shape=jax.ShapeDtypeStruct((NROW, H), jnp.float32),
    )
    tc3 = pl.pallas_call(
        _tc3_body,
        out_shape=jax.ShapeDtypeStruct((NROW // 8, 8 * C), jnp.float32),
    )

    @jax.jit
    def run(x, src, dst, w1, b1, w2, b2):
        zero_acc = jnp.zeros((NROW, H), jnp.float32)
        h = tch(x, w1)
        dacc = deg_kernel(dst, zero_acc)
        aacc1, dinv = gs1_kernel(h, dacc, src, dst, zero_acc)
        aacc2, y2 = gs2_kernel(h, dinv, aacc1, src, dst, b1, zero_acc)
        l = jnp.arange(8 * H)[:, None]
        j = jnp.arange(8 * C)[None, :]
        w2p = jnp.where(l // H == j // C, w2[l % H, j % C], 0.0)
        b2p = jnp.tile(b2, 8)[None, :]
        outp = tc3(aacc2.reshape(NC, NROW // 8, 8 * H),
                   y2.reshape(NROW // 8, 8 * H),
                   dinv.reshape(NROW // 8, 8 * H), w2p, b2p)
        return outp.reshape(NROW, C)[:N]

    return run


def kernel(x, edge_index, W1, b1, W2, b2):
    e_total = edge_index.shape[1]
    return _build(e_total)(x, edge_index[0], edge_index[1], W1, b1, W2, b2)
